# Initial kernel scaffold; baseline (speedup 1.0000x reference)
import jax
import jax.numpy as jnp
from jax.experimental import pallas as pl


def kernel(x, edge_index, batch, W1a, b1a, W1b, b1b, W2a, b2a, W2b, b2b):
    raise NotImplementedError("write your pallas kernel here")


# speedup vs baseline: 3.4648x; 3.4648x over previous
import functools

import jax
import jax.numpy as jnp
from jax import lax
from jax.experimental import pallas as pl
from jax.experimental.pallas import tpu as pltpu
from jax.experimental.pallas import tpu_sc as plsc

N = 10000
D = 128
E = 320000
G = 64

NC = 2
NS = 16
NW = NC * NS
CH = 128
NCH = 80
EPW = NCH * CH
EPAD = NW * EPW
NPAD = 10240
RPS = NPAD // NS
ZC = 128
ZCH = RPS // ZC


@functools.lru_cache(maxsize=1)
def _make_agg():
    mesh = plsc.VectorSubcoreMesh(core_axis_name="c", subcore_axis_name="s",
                                  num_cores=NC, num_subcores=NS)

    @functools.partial(
        pl.kernel,
        mesh=mesh,
        out_type=jax.ShapeDtypeStruct((NC, NPAD, D), jnp.float32),
        scratch_types=[
            pltpu.VMEM_SHARED((NPAD, D), jnp.float32),
            pltpu.VMEM((NCH, CH), jnp.int32),
            pltpu.VMEM((NCH, CH), jnp.int32),
            pltpu.VMEM((CH, D), jnp.float32),
            pltpu.SemaphoreType.DMA,
        ],
    )
    def agg(x_hbm, src_hbm, dst_hbm, out_hbm, acc, srcs, dsts, rows, sem):
        cid = lax.axis_index("c")
        sid = lax.axis_index("s")
        wid = cid * NS + sid

        def zrow(r, carry):
            def zcol(c, carry2):
                rows[r, pl.ds(c * 16, 16)] = jnp.zeros((16,), jnp.float32)
                return carry2
            return lax.fori_loop(0, D // 16, zcol, carry)
        lax.fori_loop(0, CH, zrow, 0)

        def zstripe(k, carry):
            pltpu.sync_copy(rows, acc.at[pl.ds(sid * RPS + k * ZC, ZC)])
            return carry
        lax.fori_loop(0, ZCH, zstripe, 0)

        pltpu.sync_copy(src_hbm.at[wid], srcs)
        pltpu.sync_copy(dst_hbm.at[wid], dsts)
        plsc.subcore_barrier()

        def body(j, carry):
            pltpu.async_copy(x_hbm.at[srcs.at[j]], rows, sem).wait()
            pltpu.sync_copy(rows, acc.at[dsts.at[j]], add=True)
            return carry
        lax.fori_loop(0, NCH, body, 0)

        plsc.subcore_barrier()

        def wstripe(k, carry):
            r = sid * RPS + k * ZC
            pltpu.sync_copy(acc.at[pl.ds(r, ZC)], out_hbm.at[cid, pl.ds(r, ZC)])
            return carry
        lax.fori_loop(0, ZCH, wstripe, 0)

    return agg


def _agg(x, src, dst):
    return _make_agg()(x, src, dst)


_BR = 1000


def _mlp1_body(x_ref, a_ref, wa_ref, ba_ref, wb_ref, bb_ref, o_ref):
    h = x_ref[...] + a_ref[0] + a_ref[1]
    h = jnp.dot(h, wa_ref[...], preferred_element_type=jnp.float32) + ba_ref[...]
    h = jnp.maximum(h, 0.0)
    h = jnp.dot(h, wb_ref[...], preferred_element_type=jnp.float32) + bb_ref[...]
    o_ref[...] = jnp.maximum(h, 0.0)


def _mlp1(x, agg, W1a, b1a, W1b, b1b):
    return pl.pallas_call(
        _mlp1_body,
        grid=(N // _BR,),
        in_specs=[
            pl.BlockSpec((_BR, D), lambda i: (i, 0)),
            pl.BlockSpec((NC, _BR, D), lambda i: (0, i, 0)),
            pl.BlockSpec((D, D), lambda i: (0, 0)),
            pl.BlockSpec((1, D), lambda i: (0, 0)),
            pl.BlockSpec((D, D), lambda i: (0, 0)),
            pl.BlockSpec((1, D), lambda i: (0, 0)),
        ],
        out_specs=pl.BlockSpec((_BR, D), lambda i: (i, 0)),
        out_shape=jax.ShapeDtypeStruct((N, D), jnp.float32),
    )(x, agg, W1a, b1a.reshape(1, D), W1b, b1b.reshape(1, D))


def _mlp2_pool_body(h_ref, a_ref, wa_ref, ba_ref, wb_ref, bb_ref, b_ref,
                    o_ref, acc, cnt):
    i = pl.program_id(0)
    h = h_ref[...] + a_ref[0] + a_ref[1]
    h = jnp.dot(h, wa_ref[...], preferred_element_type=jnp.float32) + ba_ref[...]
    h = jnp.maximum(h, 0.0)
    h = jnp.dot(h, wb_ref[...], preferred_element_type=jnp.float32) + bb_ref[...]
    bids = b_ref[0]
    onehot = (lax.broadcasted_iota(jnp.int32, (G, _BR), 0)
              == jnp.broadcast_to(bids, (G, _BR))).astype(jnp.float32)
    psum = jnp.dot(onehot, h, preferred_element_type=jnp.float32)

    @pl.when(i == 0)
    def _init():
        acc[...] = jnp.zeros((G, D), jnp.float32)
        cnt[...] = jnp.zeros((G, D), jnp.float32)

    acc[...] += psum
    cnt[...] += jnp.broadcast_to(
        jnp.sum(onehot, axis=1, keepdims=True), (G, D))

    @pl.when(i == (N // _BR) - 1)
    def _fin():
        o_ref[...] = acc[...] / jnp.maximum(cnt[...], 1.0)


def _mlp2_pool(h1, agg, W2a, b2a, W2b, b2b, batch3):
    return pl.pallas_call(
        _mlp2_pool_body,
        grid=(N // _BR,),
        in_specs=[
            pl.BlockSpec((_BR, D), lambda i: (i, 0)),
            pl.BlockSpec((NC, _BR, D), lambda i: (0, i, 0)),
            pl.BlockSpec((D, D), lambda i: (0, 0)),
            pl.BlockSpec((1, D), lambda i: (0, 0)),
            pl.BlockSpec((D, D), lambda i: (0, 0)),
            pl.BlockSpec((1, D), lambda i: (0, 0)),
            pl.BlockSpec((1, 1, _BR), lambda i: (i, 0, 0)),
        ],
        out_specs=pl.BlockSpec((G, D), lambda i: (0, 0)),
        out_shape=jax.ShapeDtypeStruct((G, D), jnp.float32),
        scratch_shapes=[
            pltpu.VMEM((G, D), jnp.float32),
            pltpu.VMEM((G, D), jnp.float32),
        ],
    )(h1, agg, W2a, b2a.reshape(1, D), W2b, b2b.reshape(1, D), batch3)


def kernel(x, edge_index, batch, W1a, b1a, W1b, b1b, W2a, b2a, W2b, b2b):
    pad = EPAD - E
    src = jnp.concatenate(
        [edge_index[0].astype(jnp.int32), jnp.zeros((pad,), jnp.int32)]
    ).reshape(NW, NCH, CH)
    dst = jnp.concatenate(
        [edge_index[1].astype(jnp.int32), jnp.full((pad,), N, jnp.int32)]
    ).reshape(NW, NCH, CH)
    batch3 = batch.astype(jnp.int32).reshape(N // _BR, 1, _BR)

    agg1 = _agg(x, src, dst)
    h1 = _mlp1(x, agg1, W1a, b1a, W1b, b1b)
    agg2 = _agg(h1, src, dst)
    pooled = _mlp2_pool(h1, agg2, W2a, b2a, W2b, b2b, batch3)
    return pooled.reshape(-1)


# speedup vs baseline: 3.7585x; 1.0847x over previous
import functools

import jax
import jax.numpy as jnp
from jax import lax
from jax.experimental import pallas as pl
from jax.experimental.pallas import tpu as pltpu
from jax.experimental.pallas import tpu_sc as plsc

N = 10000
D = 128
E = 320000
G = 64

NC = 2
NS = 16
NW = NC * NS
CH = 128
NCH = 80
SEG = 40
EPW = NCH * CH
EPAD = NW * EPW
NPAD = 10240
RPS = NPAD // NS
ZC = 128
ZCH = RPS // ZC


@functools.lru_cache(maxsize=1)
def _make_agg():
    mesh = plsc.VectorSubcoreMesh(core_axis_name="c", subcore_axis_name="s",
                                  num_cores=NC, num_subcores=NS)

    @functools.partial(
        pl.kernel,
        mesh=mesh,
        out_type=jax.ShapeDtypeStruct((NC, NPAD, D), jnp.float32),
        scratch_types=[
            pltpu.VMEM_SHARED((NPAD, D), jnp.float32),
            pltpu.VMEM((SEG, CH), jnp.int32),
            pltpu.VMEM((SEG, CH), jnp.int32),
            pltpu.VMEM((CH, D), jnp.float32),
            pltpu.VMEM((CH, D), jnp.float32),
            pltpu.SemaphoreType.DMA,
            pltpu.SemaphoreType.DMA,
            pltpu.SemaphoreType.DMA,
            pltpu.SemaphoreType.DMA,
        ],
    )
    def agg(x_hbm, src_hbm, dst_hbm, out_hbm, acc, srcs, dsts,
            buf0, buf1, g0, g1, s0, s1):
        cid = lax.axis_index("c")
        sid = lax.axis_index("s")
        wid = cid * NS + sid

        def zrow(r, carry):
            def zcol(c, carry2):
                buf0[r, pl.ds(c * 16, 16)] = jnp.zeros((16,), jnp.float32)
                return carry2
            return lax.fori_loop(0, D // 16, zcol, carry)
        lax.fori_loop(0, CH, zrow, 0)

        def zstripe(k, carry):
            pltpu.sync_copy(buf0, acc.at[pl.ds(sid * RPS + k * CH, CH)])
            return carry
        lax.fori_loop(0, RPS // CH, zstripe, 0)

        T = SEG // 2
        for seg in range(NCH // SEG):
            pltpu.sync_copy(src_hbm.at[wid, pl.ds(seg * SEG, SEG)], srcs)
            pltpu.sync_copy(dst_hbm.at[wid, pl.ds(seg * SEG, SEG)], dsts)
            pltpu.async_copy(x_hbm.at[srcs.at[0]], buf0, g0)

            def body(t, carry):
                j0 = 2 * t
                j1 = 2 * t + 1
                pltpu.make_async_copy(x_hbm.at[srcs.at[j0]], buf0, g0).wait()
                pltpu.async_copy(buf0, acc.at[dsts.at[j0]], s0, add=True)

                @pl.when(t > 0)
                def _():
                    pltpu.make_async_copy(buf1, acc.at[dsts.at[j1]], s1).wait()
                pltpu.async_copy(x_hbm.at[srcs.at[j1]], buf1, g1)
                pltpu.make_async_copy(x_hbm.at[srcs.at[j1]], buf1, g1).wait()
                pltpu.async_copy(buf1, acc.at[dsts.at[j1]], s1, add=True)
                pltpu.make_async_copy(buf0, acc.at[dsts.at[j0]], s0).wait()

                @pl.when(t < T - 1)
                def _():
                    pltpu.async_copy(x_hbm.at[srcs.at[2 * t + 2]], buf0, g0)
                return carry
            lax.fori_loop(0, T, body, 0)
            pltpu.make_async_copy(buf1, acc.at[dsts.at[SEG - 1]], s1).wait()

        plsc.subcore_barrier()

        def wstripe(k, carry):
            r = sid * RPS + k * ZC
            pltpu.sync_copy(acc.at[pl.ds(r, ZC)], out_hbm.at[cid, pl.ds(r, ZC)])
            return carry
        lax.fori_loop(0, ZCH, wstripe, 0)

    return agg


def _agg(x, src, dst):
    return _make_agg()(x, src, dst)


_BR = 1000


def _mlp1_body(x_ref, a_ref, wa_ref, ba_ref, wb_ref, bb_ref, o_ref):
    h = x_ref[...] + a_ref[0] + a_ref[1]
    h = jnp.dot(h, wa_ref[...], preferred_element_type=jnp.float32) + ba_ref[...]
    h = jnp.maximum(h, 0.0)
    h = jnp.dot(h, wb_ref[...], preferred_element_type=jnp.float32) + bb_ref[...]
    o_ref[...] = jnp.maximum(h, 0.0)


def _mlp1(x, agg, W1a, b1a, W1b, b1b):
    return pl.pallas_call(
        _mlp1_body,
        grid=(N // _BR,),
        in_specs=[
            pl.BlockSpec((_BR, D), lambda i: (i, 0)),
            pl.BlockSpec((NC, _BR, D), lambda i: (0, i, 0)),
            pl.BlockSpec((D, D), lambda i: (0, 0)),
            pl.BlockSpec((1, D), lambda i: (0, 0)),
            pl.BlockSpec((D, D), lambda i: (0, 0)),
            pl.BlockSpec((1, D), lambda i: (0, 0)),
        ],
        out_specs=pl.BlockSpec((_BR, D), lambda i: (i, 0)),
        out_shape=jax.ShapeDtypeStruct((N, D), jnp.float32),
    )(x, agg, W1a, b1a.reshape(1, D), W1b, b1b.reshape(1, D))


def _mlp2_pool_body(h_ref, a_ref, wa_ref, ba_ref, wb_ref, bb_ref, b_ref,
                    o_ref, acc, cnt):
    i = pl.program_id(0)
    h = h_ref[...] + a_ref[0] + a_ref[1]
    h = jnp.dot(h, wa_ref[...], preferred_element_type=jnp.float32) + ba_ref[...]
    h = jnp.maximum(h, 0.0)
    h = jnp.dot(h, wb_ref[...], preferred_element_type=jnp.float32) + bb_ref[...]
    bids = b_ref[0]
    onehot = (lax.broadcasted_iota(jnp.int32, (G, _BR), 0)
              == jnp.broadcast_to(bids, (G, _BR))).astype(jnp.float32)
    psum = jnp.dot(onehot, h, preferred_element_type=jnp.float32)

    @pl.when(i == 0)
    def _init():
        acc[...] = jnp.zeros((G, D), jnp.float32)
        cnt[...] = jnp.zeros((G, D), jnp.float32)

    acc[...] += psum
    cnt[...] += jnp.broadcast_to(
        jnp.sum(onehot, axis=1, keepdims=True), (G, D))

    @pl.when(i == (N // _BR) - 1)
    def _fin():
        o_ref[...] = acc[...] / jnp.maximum(cnt[...], 1.0)


def _mlp2_pool(h1, agg, W2a, b2a, W2b, b2b, batch3):
    return pl.pallas_call(
        _mlp2_pool_body,
        grid=(N // _BR,),
        in_specs=[
            pl.BlockSpec((_BR, D), lambda i: (i, 0)),
            pl.BlockSpec((NC, _BR, D), lambda i: (0, i, 0)),
            pl.BlockSpec((D, D), lambda i: (0, 0)),
            pl.BlockSpec((1, D), lambda i: (0, 0)),
            pl.BlockSpec((D, D), lambda i: (0, 0)),
            pl.BlockSpec((1, D), lambda i: (0, 0)),
            pl.BlockSpec((1, 1, _BR), lambda i: (i, 0, 0)),
        ],
        out_specs=pl.BlockSpec((G, D), lambda i: (0, 0)),
        out_shape=jax.ShapeDtypeStruct((G, D), jnp.float32),
        scratch_shapes=[
            pltpu.VMEM((G, D), jnp.float32),
            pltpu.VMEM((G, D), jnp.float32),
        ],
    )(h1, agg, W2a, b2a.reshape(1, D), W2b, b2b.reshape(1, D), batch3)


def kernel(x, edge_index, batch, W1a, b1a, W1b, b1b, W2a, b2a, W2b, b2b):
    pad = EPAD - E
    src = jnp.concatenate(
        [edge_index[0].astype(jnp.int32), jnp.zeros((pad,), jnp.int32)]
    ).reshape(NW, NCH, CH)
    dst = jnp.concatenate(
        [edge_index[1].astype(jnp.int32), jnp.full((pad,), N, jnp.int32)]
    ).reshape(NW, NCH, CH)
    batch3 = batch.astype(jnp.int32).reshape(N // _BR, 1, _BR)

    agg1 = _agg(x, src, dst)
    h1 = _mlp1(x, agg1, W1a, b1a, W1b, b1b)
    agg2 = _agg(h1, src, dst)
    pooled = _mlp2_pool(h1, agg2, W2a, b2a, W2b, b2b, batch3)
    return pooled.reshape(-1)


# speedup vs baseline: 9.4534x; 2.5152x over previous
import functools

import jax
import jax.numpy as jnp
from jax import lax
from jax.experimental import pallas as pl
from jax.experimental.pallas import tpu as pltpu
from jax.experimental.pallas import tpu_sc as plsc

N = 10000
D = 128
HD = D // 2
E = 320000
G = 64

NC = 2
NS = 16
CH = 128
NCH = 160
SEG = 40
EPT = NCH * CH
EPAD = NS * EPT
NPAD = 10240
RPS = NPAD // NS
ZC = 128
ZCH = RPS // ZC


@functools.lru_cache(maxsize=1)
def _make_agg():
    mesh = plsc.VectorSubcoreMesh(core_axis_name="c", subcore_axis_name="s",
                                  num_cores=NC, num_subcores=NS)

    @functools.partial(
        pl.kernel,
        mesh=mesh,
        out_type=jax.ShapeDtypeStruct((NC, NPAD, HD), jnp.float32),
        scratch_types=[
            pltpu.VMEM_SHARED((NPAD, HD), jnp.float32),
            pltpu.VMEM_SHARED((NPAD, HD), jnp.float32),
            pltpu.VMEM((SEG, CH), jnp.int32),
            pltpu.VMEM((SEG, CH), jnp.int32),
            pltpu.VMEM((CH, HD), jnp.float32),
            pltpu.VMEM((CH, HD), jnp.float32),
            pltpu.SemaphoreType.DMA,
            pltpu.SemaphoreType.DMA,
            pltpu.SemaphoreType.DMA,
            pltpu.SemaphoreType.DMA,
        ],
    )
    def agg(x2_hbm, src_hbm, dst_hbm, out_hbm, xsp, acc, srcs, dsts,
            buf0, buf1, g0, g1, s0, s1):
        cid = lax.axis_index("c")
        sid = lax.axis_index("s")

        @pl.when(sid < 10)
        def _stage():
            pltpu.sync_copy(x2_hbm.at[cid, pl.ds(sid * 1000, 1000)],
                            xsp.at[pl.ds(sid * 1000, 1000)])

        def zrow(r, carry):
            def zcol(c, carry2):
                buf0[r, pl.ds(c * 16, 16)] = jnp.zeros((16,), jnp.float32)
                return carry2
            return lax.fori_loop(0, HD // 16, zcol, carry)
        lax.fori_loop(0, CH, zrow, 0)

        def zstripe(k, carry):
            pltpu.sync_copy(buf0, acc.at[pl.ds(sid * RPS + k * ZC, ZC)])
            return carry
        lax.fori_loop(0, ZCH, zstripe, 0)
        plsc.subcore_barrier()

        T = SEG // 2
        for seg in range(NCH // SEG):
            pltpu.sync_copy(src_hbm.at[sid, pl.ds(seg * SEG, SEG)], srcs)
            pltpu.sync_copy(dst_hbm.at[sid, pl.ds(seg * SEG, SEG)], dsts)
            pltpu.async_copy(xsp.at[srcs.at[0]], buf0, g0)

            def body(t, carry):
                j0 = 2 * t
                j1 = 2 * t + 1
                pltpu.make_async_copy(xsp.at[srcs.at[j0]], buf0, g0).wait()
                pltpu.async_copy(buf0, acc.at[dsts.at[j0]], s0, add=True)

                @pl.when(t > 0)
                def _():
                    pltpu.make_async_copy(buf1, acc.at[dsts.at[j1]], s1).wait()
                pltpu.async_copy(xsp.at[srcs.at[j1]], buf1, g1)
                pltpu.make_async_copy(xsp.at[srcs.at[j1]], buf1, g1).wait()
                pltpu.async_copy(buf1, acc.at[dsts.at[j1]], s1, add=True)
                pltpu.make_async_copy(buf0, acc.at[dsts.at[j0]], s0).wait()

                @pl.when(t < T - 1)
                def _():
                    pltpu.async_copy(xsp.at[srcs.at[2 * t + 2]], buf0, g0)
                return carry
            lax.fori_loop(0, T, body, 0)
            pltpu.make_async_copy(buf1, acc.at[dsts.at[SEG - 1]], s1).wait()

        plsc.subcore_barrier()

        def wstripe(k, carry):
            r = sid * RPS + k * ZC
            pltpu.sync_copy(acc.at[pl.ds(r, ZC)], out_hbm.at[cid, pl.ds(r, ZC)])
            return carry
        lax.fori_loop(0, ZCH, wstripe, 0)

    return agg


def _agg(x2, src, dst):
    return _make_agg()(x2, src, dst)


_BR = 1000


def _mlp1_body(x_ref, a_ref, wa_ref, ba_ref, wb_ref, bb_ref, o_ref, os_ref):
    h = x_ref[...] + jnp.concatenate([a_ref[0], a_ref[1]], axis=1)
    h = jnp.dot(h, wa_ref[...], preferred_element_type=jnp.float32) + ba_ref[...]
    h = jnp.maximum(h, 0.0)
    h = jnp.dot(h, wb_ref[...], preferred_element_type=jnp.float32) + bb_ref[...]
    h = jnp.maximum(h, 0.0)
    o_ref[...] = h
    os_ref[0] = h[:, :HD]
    os_ref[1] = h[:, HD:]


def _mlp1(x, agg, W1a, b1a, W1b, b1b):
    return pl.pallas_call(
        _mlp1_body,
        grid=(N // _BR,),
        in_specs=[
            pl.BlockSpec((_BR, D), lambda i: (i, 0)),
            pl.BlockSpec((NC, _BR, HD), lambda i: (0, i, 0)),
            pl.BlockSpec((D, D), lambda i: (0, 0)),
            pl.BlockSpec((1, D), lambda i: (0, 0)),
            pl.BlockSpec((D, D), lambda i: (0, 0)),
            pl.BlockSpec((1, D), lambda i: (0, 0)),
        ],
        out_specs=[
            pl.BlockSpec((_BR, D), lambda i: (i, 0)),
            pl.BlockSpec((NC, _BR, HD), lambda i: (0, i, 0)),
        ],
        out_shape=[
            jax.ShapeDtypeStruct((N, D), jnp.float32),
            jax.ShapeDtypeStruct((NC, N, HD), jnp.float32),
        ],
    )(x, agg, W1a, b1a.reshape(1, D), W1b, b1b.reshape(1, D))


def _mlp2_pool_body(h_ref, a_ref, wa_ref, ba_ref, wb_ref, bb_ref, b_ref,
                    o_ref, acc, cnt):
    i = pl.program_id(0)
    h = h_ref[...] + jnp.concatenate([a_ref[0], a_ref[1]], axis=1)
    h = jnp.dot(h, wa_ref[...], preferred_element_type=jnp.float32) + ba_ref[...]
    h = jnp.maximum(h, 0.0)
    h = jnp.dot(h, wb_ref[...], preferred_element_type=jnp.float32) + bb_ref[...]
    bids = b_ref[0]
    onehot = (lax.broadcasted_iota(jnp.int32, (G, _BR), 0)
              == jnp.broadcast_to(bids, (G, _BR))).astype(jnp.float32)
    psum = jnp.dot(onehot, h, preferred_element_type=jnp.float32)

    @pl.when(i == 0)
    def _init():
        acc[...] = jnp.zeros((G, D), jnp.float32)
        cnt[...] = jnp.zeros((G, D), jnp.float32)

    acc[...] += psum
    cnt[...] += jnp.broadcast_to(
        jnp.sum(onehot, axis=1, keepdims=True), (G, D))

    @pl.when(i == (N // _BR) - 1)
    def _fin():
        o_ref[...] = acc[...] / jnp.maximum(cnt[...], 1.0)


def _mlp2_pool(h1, agg, W2a, b2a, W2b, b2b, batch3):
    return pl.pallas_call(
        _mlp2_pool_body,
        grid=(N // _BR,),
        in_specs=[
            pl.BlockSpec((_BR, D), lambda i: (i, 0)),
            pl.BlockSpec((NC, _BR, HD), lambda i: (0, i, 0)),
            pl.BlockSpec((D, D), lambda i: (0, 0)),
            pl.BlockSpec((1, D), lambda i: (0, 0)),
            pl.BlockSpec((D, D), lambda i: (0, 0)),
            pl.BlockSpec((1, D), lambda i: (0, 0)),
            pl.BlockSpec((1, 1, _BR), lambda i: (i, 0, 0)),
        ],
        out_specs=pl.BlockSpec((G, D), lambda i: (0, 0)),
        out_shape=jax.ShapeDtypeStruct((G, D), jnp.float32),
        scratch_shapes=[
            pltpu.VMEM((G, D), jnp.float32),
            pltpu.VMEM((G, D), jnp.float32),
        ],
    )(h1, agg, W2a, b2a.reshape(1, D), W2b, b2b.reshape(1, D), batch3)


def kernel(x, edge_index, batch, W1a, b1a, W1b, b1b, W2a, b2a, W2b, b2b):
    pad = EPAD - E
    src = jnp.concatenate(
        [edge_index[0].astype(jnp.int32), jnp.zeros((pad,), jnp.int32)]
    ).reshape(NS, NCH, CH)
    dst = jnp.concatenate(
        [edge_index[1].astype(jnp.int32), jnp.full((pad,), N, jnp.int32)]
    ).reshape(NS, NCH, CH)
    batch3 = batch.astype(jnp.int32).reshape(N // _BR, 1, _BR)

    x2 = jnp.stack([x[:, :HD], x[:, HD:]])
    agg1 = _agg(x2, src, dst)
    h1, h1s = _mlp1(x, agg1, W1a, b1a, W1b, b1b)
    agg2 = _agg(h1s, src, dst)
    pooled = _mlp2_pool(h1, agg2, W2a, b2a, W2b, b2b, batch3)
    return pooled.reshape(-1)


# speedup vs baseline: 9.5877x; 1.0142x over previous
import functools

import jax
import jax.numpy as jnp
from jax import lax
from jax.experimental import pallas as pl
from jax.experimental.pallas import tpu as pltpu
from jax.experimental.pallas import tpu_sc as plsc

N = 10000
D = 128
HD = D // 2
E = 320000
G = 64

NC = 2
NS = 16
CH = 128
NCH = 160
SEG = 40
EPT = NCH * CH
EPAD = NS * EPT
NPAD = 10240
RPS = NPAD // NS
ZC = 128
ZCH = RPS // ZC


@functools.lru_cache(maxsize=1)
def _make_agg():
    mesh = plsc.VectorSubcoreMesh(core_axis_name="c", subcore_axis_name="s",
                                  num_cores=NC, num_subcores=NS)

    @functools.partial(
        pl.kernel,
        mesh=mesh,
        out_type=jax.ShapeDtypeStruct((NC, NPAD, HD), jnp.float32),
        scratch_types=[
            pltpu.VMEM_SHARED((NPAD, HD), jnp.float32),
            pltpu.VMEM_SHARED((NPAD, HD), jnp.float32),
            pltpu.VMEM((SEG, CH), jnp.int32),
            pltpu.VMEM((SEG, CH), jnp.int32),
            pltpu.VMEM((CH, HD), jnp.float32),
            pltpu.VMEM((CH, HD), jnp.float32),
            pltpu.SemaphoreType.DMA,
            pltpu.SemaphoreType.DMA,
            pltpu.SemaphoreType.DMA,
            pltpu.SemaphoreType.DMA,
        ],
    )
    def agg(x2_hbm, src_hbm, dst_hbm, out_hbm, xsp, acc, srcs, dsts,
            buf0, buf1, g0, g1, s0, s1):
        cid = lax.axis_index("c")
        sid = lax.axis_index("s")

        @pl.when(sid < 10)
        def _stage():
            pltpu.sync_copy(x2_hbm.at[cid, pl.ds(sid * 1000, 1000)],
                            xsp.at[pl.ds(sid * 1000, 1000)])

        def zrow(r, carry):
            def zcol(c, carry2):
                buf0[r, pl.ds(c * 16, 16)] = jnp.zeros((16,), jnp.float32)
                return carry2
            return lax.fori_loop(0, HD // 16, zcol, carry)
        lax.fori_loop(0, CH, zrow, 0)

        def zstripe(k, carry):
            pltpu.sync_copy(buf0, acc.at[pl.ds(sid * RPS + k * ZC, ZC)])
            return carry
        lax.fori_loop(0, ZCH, zstripe, 0)
        plsc.subcore_barrier()

        T = SEG // 2
        for seg in range(NCH // SEG):
            pltpu.sync_copy(src_hbm.at[sid, pl.ds(seg * SEG, SEG)], srcs)
            pltpu.sync_copy(dst_hbm.at[sid, pl.ds(seg * SEG, SEG)], dsts)
            pltpu.async_copy(xsp.at[srcs.at[0]], buf0, g0)

            def body(t, carry):
                j0 = 2 * t
                j1 = 2 * t + 1
                pltpu.make_async_copy(xsp.at[srcs.at[j0]], buf0, g0).wait()
                pltpu.async_copy(buf0, acc.at[dsts.at[j0]], s0, add=True)

                @pl.when(t > 0)
                def _():
                    pltpu.make_async_copy(buf1, acc.at[dsts.at[j1]], s1).wait()
                pltpu.async_copy(xsp.at[srcs.at[j1]], buf1, g1)
                pltpu.make_async_copy(xsp.at[srcs.at[j1]], buf1, g1).wait()
                pltpu.async_copy(buf1, acc.at[dsts.at[j1]], s1, add=True)
                pltpu.make_async_copy(buf0, acc.at[dsts.at[j0]], s0).wait()

                @pl.when(t < T - 1)
                def _():
                    pltpu.async_copy(xsp.at[srcs.at[2 * t + 2]], buf0, g0)
                return carry
            lax.fori_loop(0, T, body, 0)
            pltpu.make_async_copy(buf1, acc.at[dsts.at[SEG - 1]], s1).wait()

        plsc.subcore_barrier()

        r = sid * RPS
        pltpu.sync_copy(acc.at[pl.ds(r, RPS)], out_hbm.at[cid, pl.ds(r, RPS)])

    return agg


def _agg(x2, src, dst):
    return _make_agg()(x2, src, dst)


_BR = 2000


def _mlp1_body(x_ref, a_ref, wa_ref, ba_ref, wb_ref, bb_ref, os_ref):
    h = x_ref[...] + jnp.concatenate([a_ref[0], a_ref[1]], axis=1)
    h = jnp.dot(h, wa_ref[...], preferred_element_type=jnp.float32) + ba_ref[...]
    h = jnp.maximum(h, 0.0)
    h = jnp.dot(h, wb_ref[...], preferred_element_type=jnp.float32) + bb_ref[...]
    h = jnp.maximum(h, 0.0)
    os_ref[0] = h[:, :HD]
    os_ref[1] = h[:, HD:]


def _mlp1(x, agg, W1a, b1a, W1b, b1b):
    return pl.pallas_call(
        _mlp1_body,
        grid=(N // _BR,),
        in_specs=[
            pl.BlockSpec((_BR, D), lambda i: (i, 0)),
            pl.BlockSpec((NC, _BR, HD), lambda i: (0, i, 0)),
            pl.BlockSpec((D, D), lambda i: (0, 0)),
            pl.BlockSpec((1, D), lambda i: (0, 0)),
            pl.BlockSpec((D, D), lambda i: (0, 0)),
            pl.BlockSpec((1, D), lambda i: (0, 0)),
        ],
        out_specs=pl.BlockSpec((NC, _BR, HD), lambda i: (0, i, 0)),
        out_shape=jax.ShapeDtypeStruct((NC, N, HD), jnp.float32),
    )(x, agg, W1a, b1a.reshape(1, D), W1b, b1b.reshape(1, D))


def _mlp2_pool_body(h_ref, a_ref, wa_ref, ba_ref, wb_ref, bb_ref, b_ref,
                    o_ref, acc, cnt):
    i = pl.program_id(0)
    h = (jnp.concatenate([h_ref[0], h_ref[1]], axis=1)
         + jnp.concatenate([a_ref[0], a_ref[1]], axis=1))
    h = jnp.dot(h, wa_ref[...], preferred_element_type=jnp.float32) + ba_ref[...]
    h = jnp.maximum(h, 0.0)
    h = jnp.dot(h, wb_ref[...], preferred_element_type=jnp.float32) + bb_ref[...]
    bids = b_ref[0]
    onehot = (lax.broadcasted_iota(jnp.int32, (G, _BR), 0)
              == jnp.broadcast_to(bids, (G, _BR))).astype(jnp.float32)
    psum = jnp.dot(onehot, h, preferred_element_type=jnp.float32)

    @pl.when(i == 0)
    def _init():
        acc[...] = jnp.zeros((G, D), jnp.float32)
        cnt[...] = jnp.zeros((G, D), jnp.float32)

    acc[...] += psum
    cnt[...] += jnp.broadcast_to(
        jnp.sum(onehot, axis=1, keepdims=True), (G, D))

    @pl.when(i == (N // _BR) - 1)
    def _fin():
        o_ref[...] = acc[...] / jnp.maximum(cnt[...], 1.0)


def _mlp2_pool(h1s, agg, W2a, b2a, W2b, b2b, batch3):
    return pl.pallas_call(
        _mlp2_pool_body,
        grid=(N // _BR,),
        in_specs=[
            pl.BlockSpec((NC, _BR, HD), lambda i: (0, i, 0)),
            pl.BlockSpec((NC, _BR, HD), lambda i: (0, i, 0)),
            pl.BlockSpec((D, D), lambda i: (0, 0)),
            pl.BlockSpec((1, D), lambda i: (0, 0)),
            pl.BlockSpec((D, D), lambda i: (0, 0)),
            pl.BlockSpec((1, D), lambda i: (0, 0)),
            pl.BlockSpec((1, 1, _BR), lambda i: (i, 0, 0)),
        ],
        out_specs=pl.BlockSpec((G, D), lambda i: (0, 0)),
        out_shape=jax.ShapeDtypeStruct((G, D), jnp.float32),
        scratch_shapes=[
            pltpu.VMEM((G, D), jnp.float32),
            pltpu.VMEM((G, D), jnp.float32),
        ],
    )(h1s, agg, W2a, b2a.reshape(1, D), W2b, b2b.reshape(1, D), batch3)


def kernel(x, edge_index, batch, W1a, b1a, W1b, b1b, W2a, b2a, W2b, b2b):
    pad = EPAD - E
    src = jnp.concatenate(
        [edge_index[0].astype(jnp.int32), jnp.zeros((pad,), jnp.int32)]
    ).reshape(NS, NCH, CH)
    dst = jnp.concatenate(
        [edge_index[1].astype(jnp.int32), jnp.full((pad,), N, jnp.int32)]
    ).reshape(NS, NCH, CH)
    batch3 = batch.astype(jnp.int32).reshape(N // _BR, 1, _BR)

    x2 = jnp.stack([x[:, :HD], x[:, HD:]])
    agg1 = _agg(x2, src, dst)
    h1s = _mlp1(x, agg1, W1a, b1a, W1b, b1b)
    agg2 = _agg(h1s, src, dst)
    pooled = _mlp2_pool(h1s, agg2, W2a, b2a, W2b, b2b, batch3)
    return pooled.reshape(-1)


# speedup vs baseline: 9.6237x; 1.0038x over previous
import functools

import jax
import jax.numpy as jnp
from jax import lax
from jax.experimental import pallas as pl
from jax.experimental.pallas import tpu as pltpu
from jax.experimental.pallas import tpu_sc as plsc

N = 10000
D = 128
HD = D // 2
E = 320000
G = 64

NC = 2
NS = 16
CH = 125
NCH = 160
SEG = 40
NPAD = 10240
RPS = NPAD // NS
ZC = 128
ZCH = RPS // ZC


@functools.lru_cache(maxsize=1)
def _make_agg():
    mesh = plsc.VectorSubcoreMesh(core_axis_name="c", subcore_axis_name="s",
                                  num_cores=NC, num_subcores=NS)

    @functools.partial(
        pl.kernel,
        mesh=mesh,
        out_type=jax.ShapeDtypeStruct((NC, NPAD, HD), jnp.float32),
        scratch_types=[
            pltpu.VMEM_SHARED((NPAD, HD), jnp.float32),
            pltpu.VMEM_SHARED((NPAD, HD), jnp.float32),
            pltpu.VMEM((SEG, CH), jnp.int32),
            pltpu.VMEM((SEG, CH), jnp.int32),
            pltpu.VMEM((ZC, HD), jnp.float32),
            pltpu.VMEM((ZC, HD), jnp.float32),
            pltpu.SemaphoreType.DMA,
            pltpu.SemaphoreType.DMA,
            pltpu.SemaphoreType.DMA,
            pltpu.SemaphoreType.DMA,
        ],
    )
    def agg(x2_hbm, src_hbm, dst_hbm, out_hbm, xsp, acc, srcs, dsts,
            buf0, buf1, g0, g1, s0, s1):
        cid = lax.axis_index("c")
        sid = lax.axis_index("s")

        @pl.when(sid < 10)
        def _stage():
            pltpu.sync_copy(x2_hbm.at[cid, pl.ds(sid * 1000, 1000)],
                            xsp.at[pl.ds(sid * 1000, 1000)])

        def zrow(r, carry):
            def zcol(c, carry2):
                buf0[r, pl.ds(c * 16, 16)] = jnp.zeros((16,), jnp.float32)
                return carry2
            return lax.fori_loop(0, HD // 16, zcol, carry)
        lax.fori_loop(0, ZC, zrow, 0)

        def zstripe(k, carry):
            pltpu.sync_copy(buf0, acc.at[pl.ds(sid * RPS + k * ZC, ZC)])
            return carry
        lax.fori_loop(0, ZCH, zstripe, 0)
        plsc.subcore_barrier()

        b0 = buf0.at[pl.ds(0, CH)]
        b1 = buf1.at[pl.ds(0, CH)]

        T = SEG // 2
        for seg in range(NCH // SEG):
            pltpu.sync_copy(src_hbm.at[sid, pl.ds(seg * SEG, SEG)], srcs)
            pltpu.sync_copy(dst_hbm.at[sid, pl.ds(seg * SEG, SEG)], dsts)
            pltpu.async_copy(xsp.at[srcs.at[0]], b0, g0)

            def body(t, carry):
                j0 = 2 * t
                j1 = 2 * t + 1
                pltpu.make_async_copy(xsp.at[srcs.at[j0]], b0, g0).wait()
                pltpu.async_copy(b0, acc.at[dsts.at[j0]], s0, add=True)

                @pl.when(t > 0)
                def _():
                    pltpu.make_async_copy(b1, acc.at[dsts.at[j1]], s1).wait()
                pltpu.async_copy(xsp.at[srcs.at[j1]], b1, g1)
                pltpu.make_async_copy(xsp.at[srcs.at[j1]], b1, g1).wait()
                pltpu.async_copy(b1, acc.at[dsts.at[j1]], s1, add=True)
                pltpu.make_async_copy(b0, acc.at[dsts.at[j0]], s0).wait()

                @pl.when(t < T - 1)
                def _():
                    pltpu.async_copy(xsp.at[srcs.at[2 * t + 2]], b0, g0)
                return carry
            lax.fori_loop(0, T, body, 0)
            pltpu.make_async_copy(b1, acc.at[dsts.at[SEG - 1]], s1).wait()

        plsc.subcore_barrier()

        r = sid * RPS
        pltpu.sync_copy(acc.at[pl.ds(r, RPS)], out_hbm.at[cid, pl.ds(r, RPS)])

    return agg


def _agg(x2, src, dst):
    return _make_agg()(x2, src, dst)


_BR = 2000


def _mlp1_body(x_ref, a_ref, wa_ref, ba_ref, wb_ref, bb_ref, os_ref):
    h = x_ref[...] + jnp.concatenate([a_ref[0], a_ref[1]], axis=1)
    h = jnp.dot(h, wa_ref[...], preferred_element_type=jnp.float32) + ba_ref[...]
    h = jnp.maximum(h, 0.0)
    h = jnp.dot(h, wb_ref[...], preferred_element_type=jnp.float32) + bb_ref[...]
    h = jnp.maximum(h, 0.0)
    os_ref[0] = h[:, :HD]
    os_ref[1] = h[:, HD:]


def _mlp1(x, agg, W1a, b1a, W1b, b1b):
    return pl.pallas_call(
        _mlp1_body,
        grid=(N // _BR,),
        in_specs=[
            pl.BlockSpec((_BR, D), lambda i: (i, 0)),
            pl.BlockSpec((NC, _BR, HD), lambda i: (0, i, 0)),
            pl.BlockSpec((D, D), lambda i: (0, 0)),
            pl.BlockSpec((1, D), lambda i: (0, 0)),
            pl.BlockSpec((D, D), lambda i: (0, 0)),
            pl.BlockSpec((1, D), lambda i: (0, 0)),
        ],
        out_specs=pl.BlockSpec((NC, _BR, HD), lambda i: (0, i, 0)),
        out_shape=jax.ShapeDtypeStruct((NC, N, HD), jnp.float32),
    )(x, agg, W1a, b1a.reshape(1, D), W1b, b1b.reshape(1, D))


def _mlp2_pool_body(h_ref, a_ref, wa_ref, ba_ref, wb_ref, bb_ref, b_ref,
                    o_ref, acc, cnt):
    i = pl.program_id(0)
    h = (jnp.concatenate([h_ref[0], h_ref[1]], axis=1)
         + jnp.concatenate([a_ref[0], a_ref[1]], axis=1))
    h = jnp.dot(h, wa_ref[...], preferred_element_type=jnp.float32) + ba_ref[...]
    h = jnp.maximum(h, 0.0)
    h = jnp.dot(h, wb_ref[...], preferred_element_type=jnp.float32) + bb_ref[...]
    bids = b_ref[0]
    onehot = (lax.broadcasted_iota(jnp.int32, (G, _BR), 0)
              == jnp.broadcast_to(bids, (G, _BR))).astype(jnp.float32)
    psum = jnp.dot(onehot, h, preferred_element_type=jnp.float32)

    @pl.when(i == 0)
    def _init():
        acc[...] = jnp.zeros((G, D), jnp.float32)
        cnt[...] = jnp.zeros((G, D), jnp.float32)

    acc[...] += psum
    cnt[...] += jnp.broadcast_to(
        jnp.sum(onehot, axis=1, keepdims=True), (G, D))

    @pl.when(i == (N // _BR) - 1)
    def _fin():
        o_ref[...] = acc[...] / jnp.maximum(cnt[...], 1.0)


def _mlp2_pool(h1s, agg, W2a, b2a, W2b, b2b, batch3):
    return pl.pallas_call(
        _mlp2_pool_body,
        grid=(N // _BR,),
        in_specs=[
            pl.BlockSpec((NC, _BR, HD), lambda i: (0, i, 0)),
            pl.BlockSpec((NC, _BR, HD), lambda i: (0, i, 0)),
            pl.BlockSpec((D, D), lambda i: (0, 0)),
            pl.BlockSpec((1, D), lambda i: (0, 0)),
            pl.BlockSpec((D, D), lambda i: (0, 0)),
            pl.BlockSpec((1, D), lambda i: (0, 0)),
            pl.BlockSpec((1, 1, _BR), lambda i: (i, 0, 0)),
        ],
        out_specs=pl.BlockSpec((G, D), lambda i: (0, 0)),
        out_shape=jax.ShapeDtypeStruct((G, D), jnp.float32),
        scratch_shapes=[
            pltpu.VMEM((G, D), jnp.float32),
            pltpu.VMEM((G, D), jnp.float32),
        ],
    )(h1s, agg, W2a, b2a.reshape(1, D), W2b, b2b.reshape(1, D), batch3)


def kernel(x, edge_index, batch, W1a, b1a, W1b, b1b, W2a, b2a, W2b, b2b):
    src = edge_index[0].astype(jnp.int32).reshape(NS, NCH, CH)
    dst = edge_index[1].astype(jnp.int32).reshape(NS, NCH, CH)
    batch3 = batch.astype(jnp.int32).reshape(N // _BR, 1, _BR)

    x2 = jnp.stack([x[:, :HD], x[:, HD:]])
    agg1 = _agg(x2, src, dst)
    h1s = _mlp1(x, agg1, W1a, b1a, W1b, b1b)
    agg2 = _agg(h1s, src, dst)
    pooled = _mlp2_pool(h1s, agg2, W2a, b2a, W2b, b2b, batch3)
    return pooled.reshape(-1)
